# Initial kernel scaffold; baseline (speedup 1.0000x reference)
import jax
import jax.numpy as jnp
from jax.experimental import pallas as pl


def kernel(user_indices, item_indices, adj_row, adj_col, user_table, item_table, temporal, W0, b0, W1, b1, Wa1, ba1, Wa2, ba2, Wp1, bp1, Wp2, bp2):
    raise NotImplementedError("write your pallas kernel here")


# speedup vs baseline: 15.8440x; 15.8440x over previous
import functools

import jax
import jax.numpy as jnp
from jax import lax
from jax.experimental import pallas as pl
from jax.experimental.pallas import tpu as pltpu
from jax.experimental.pallas import tpu_sc as plsc

NU_C = 25000
NI_C = 25000
N_C = NU_C + NI_C
D_C = 64
T_C = 3
B_C = 4096

NC = 2
NS = 16
G = 128
K = 4

NP = 50048
RPT = NP // NS


def _mesh():
    return plsc.VectorSubcoreMesh(
        core_axis_name="c", subcore_axis_name="s", num_cores=NC, num_subcores=NS
    )


def _deg_body(gpt, n_chunk, rowg, degp, rowbuf, ones, zbuf, acc, sem):
    c = lax.axis_index("c")
    s = lax.axis_index("s")
    z16 = jnp.zeros((16,), jnp.float32)
    o16 = jnp.ones((16,), jnp.float32)

    def fill(i, _):
        zbuf[pl.ds(i * 16, 16)] = z16
        return 0

    lax.fori_loop(0, 64, fill, 0)
    for i in range(G // 16):
        ones[pl.ds(i * 16, 16)] = o16
    base = s * RPT
    for off in range(0, RPT, 1024):
        w = min(1024, RPT - off)
        pltpu.sync_copy(zbuf.at[pl.ds(0, w)], acc.at[pl.ds(base + off, w)])
    plsc.subcore_barrier()

    g0 = (c * NS + s) * gpt

    def chunk(i, _):
        pltpu.sync_copy(rowg.at[pl.ds(g0 + i * K, K)], rowbuf)
        descs = [
            pltpu.async_copy(ones.at[pl.ds(0, G)], acc.at[rowbuf.at[j]], sem,
                             add=True)
            for j in range(K)
        ]
        for dsc in descs:
            dsc.wait()
        return 0

    lax.fori_loop(0, n_chunk, chunk, 0)
    plsc.subcore_barrier()
    wo = 3072
    pltpu.sync_copy(acc.at[pl.ds(s * wo, wo)],
                    degp.at[pl.ds(c * NP + s * wo, wo)])
    tail = NP - NS * wo
    if tail:
        @pl.when(s == NS - 1)
        def _():
            pltpu.sync_copy(acc.at[pl.ds(NS * wo, tail)],
                            degp.at[pl.ds(c * NP + NS * wo, tail)])


def _sc_deg(rowg):
    gtot = rowg.shape[0]
    gpt = gtot // (NC * NS)
    body = functools.partial(_deg_body, gpt, gpt // K)
    return pl.kernel(
        body,
        out_type=jax.ShapeDtypeStruct((NC * NP,), jnp.float32),
        mesh=_mesh(),
        compiler_params=pltpu.CompilerParams(use_tc_tiling_on_sc=False),
        scratch_types=[
            pltpu.VMEM((K, G), jnp.int32),
            pltpu.VMEM((G,), jnp.float32),
            pltpu.VMEM((1024,), jnp.float32),
            pltpu.VMEM_SHARED((NP,), jnp.float32),
            pltpu.SemaphoreType.DMA,
        ],
    )(rowg)


def _spmm_body(tc, gpt, n_chunk, colg, rowg, q, out,
               colbuf, rowbuf, gbuf, acc, semg, sems):
    c = lax.axis_index("c")
    s = lax.axis_index("s")
    z16 = jnp.zeros((16,), jnp.float32)

    def fill(i, _):
        gbuf[i, pl.ds(0, 16)] = z16
        gbuf[i, pl.ds(16, 16)] = z16
        return 0

    lax.fori_loop(0, K * G, fill, 0)
    base = s * RPT
    g0 = s * gpt

    for t in range(tc):
        for off in range(0, RPT, K * G):
            w = min(K * G, RPT - off)
            pltpu.sync_copy(gbuf.at[pl.ds(0, w)], acc.at[pl.ds(base + off, w)])
        plsc.subcore_barrier()

        def chunk(i, _):
            pltpu.sync_copy(colg.at[pl.ds(g0 + i * K, K)], colbuf)
            pltpu.sync_copy(rowg.at[pl.ds(g0 + i * K, K)], rowbuf)
            gd = [
                pltpu.async_copy(q.at[t, c].at[colbuf.at[j]],
                                 gbuf.at[pl.ds(j * G, G)], semg)
                for j in range(K)
            ]
            for d in gd:
                d.wait()
            sd = [
                pltpu.async_copy(gbuf.at[pl.ds(j * G, G)],
                                 acc.at[rowbuf.at[j]], sems, add=True)
                for j in range(K)
            ]
            for d in sd:
                d.wait()
            return 0

        lax.fori_loop(0, n_chunk, chunk, 0)
        plsc.subcore_barrier()
        pltpu.sync_copy(acc.at[pl.ds(base, RPT)],
                        out.at[t, c, pl.ds(base, RPT)])
        if t + 1 < tc:
            lax.fori_loop(0, K * G, fill, 0)
            plsc.subcore_barrier()


def _sc_spmm(colg, rowg, q):
    tc = q.shape[0]
    gtot = colg.shape[0]
    gpt = gtot // NS
    body = functools.partial(_spmm_body, tc, gpt, gpt // K)
    return pl.kernel(
        body,
        out_type=jax.ShapeDtypeStruct((tc, NC, NP, 32), jnp.float32),
        mesh=_mesh(),
        compiler_params=pltpu.CompilerParams(use_tc_tiling_on_sc=False),
        scratch_types=[
            pltpu.VMEM((K, G), jnp.int32),
            pltpu.VMEM((K, G), jnp.int32),
            pltpu.VMEM((K * G, 32), jnp.float32),
            pltpu.VMEM_SHARED((NP, 32), jnp.float32),
            pltpu.SemaphoreType.DMA,
            pltpu.SemaphoreType.DMA,
        ],
    )(colg, rowg, q)


def _gather_body(ui, ii, v2, dinv, ut, it,
                 gu, gi, du, di, ue, ie,
                 ubuf, ibuf, vbuf, ebuf, dbuf, sem):
    c = lax.axis_index("c")
    s = lax.axis_index("s")
    w = s * NC + c
    bpw = B_C // (NC * NS)
    b0 = w * bpw
    pltpu.sync_copy(ui.at[pl.ds(b0, bpw)], ubuf.at[0])
    pltpu.sync_copy(ii.at[pl.ds(b0, bpw)], ibuf.at[0])
    pltpu.async_copy(ut.at[ubuf.at[0]], ebuf, sem).wait()
    pltpu.sync_copy(ebuf, ue.at[pl.ds(b0, bpw)])
    pltpu.async_copy(it.at[ibuf.at[0]], ebuf, sem).wait()
    pltpu.sync_copy(ebuf, ie.at[pl.ds(b0, bpw)])
    pltpu.async_copy(dinv.at[ubuf.at[0]], dbuf, sem).wait()
    pltpu.sync_copy(dbuf, du.at[pl.ds(b0, bpw)])
    nu16 = jnp.full((16,), NU_C, jnp.int32)
    for j in range(bpw // 16):
        ibuf[0, pl.ds(j * 16, 16)] = ibuf[0, pl.ds(j * 16, 16)] + nu16
    pltpu.async_copy(dinv.at[ibuf.at[0]], dbuf, sem).wait()
    pltpu.sync_copy(dbuf, di.at[pl.ds(b0, bpw)])
    for t in range(T_C):
        for h in range(2):
            pltpu.async_copy(v2.at[t, h].at[ubuf.at[0]], vbuf, sem).wait()
            pltpu.sync_copy(vbuf, gu.at[t, h, pl.ds(b0, bpw)])
            pltpu.async_copy(v2.at[t, h].at[ibuf.at[0]], vbuf, sem).wait()
            pltpu.sync_copy(vbuf, gi.at[t, h, pl.ds(b0, bpw)])


def _sc_gather(ui, ii, v2, dinv, ut, it):
    bpw = B_C // (NC * NS)
    return pl.kernel(
        _gather_body,
        out_type=(
            jax.ShapeDtypeStruct((T_C, NC, B_C, 32), jnp.float32),
            jax.ShapeDtypeStruct((T_C, NC, B_C, 32), jnp.float32),
            jax.ShapeDtypeStruct((B_C, 8), jnp.float32),
            jax.ShapeDtypeStruct((B_C, 8), jnp.float32),
            jax.ShapeDtypeStruct((B_C, D_C), jnp.float32),
            jax.ShapeDtypeStruct((B_C, D_C), jnp.float32),
        ),
        mesh=_mesh(),
        compiler_params=pltpu.CompilerParams(use_tc_tiling_on_sc=False),
        scratch_types=[
            pltpu.VMEM((1, bpw), jnp.int32),
            pltpu.VMEM((1, bpw), jnp.int32),
            pltpu.VMEM((bpw, 32), jnp.float32),
            pltpu.VMEM((bpw, D_C), jnp.float32),
            pltpu.VMEM((bpw, 8), jnp.float32),
            pltpu.SemaphoreType.DMA,
        ],
    )(ui, ii, v2, dinv, ut, it)


def _prep_body(x, deg, temporal, w0, q1, dinv):
    dv1 = jnp.where(deg[...][:, :1] > 0.0, lax.rsqrt(deg[...][:, :1]), 0.0)
    dinv[...] = jnp.broadcast_to(dv1, dinv.shape)
    xw = jnp.dot(x[...], w0[...], preferred_element_type=jnp.float32,
                 precision=lax.Precision.HIGHEST)
    tw = jnp.dot(temporal[...], w0[...], preferred_element_type=jnp.float32,
                 precision=lax.Precision.HIGHEST)
    for t in range(T_C):
        qt = dv1 * (xw + tw[t][None, :])
        q1[t, 0] = qt[:, :32]
        q1[t, 1] = qt[:, 32:]


def _tc_prep(x, deg2, temporal, w0):
    nb = NP // RPT
    return pl.pallas_call(
        _prep_body,
        grid=(nb,),
        in_specs=[
            pl.BlockSpec((RPT, D_C), lambda i: (i, 0)),
            pl.BlockSpec((RPT, 1), lambda i: (i, 0)),
            pl.BlockSpec((T_C, D_C), lambda i: (0, 0)),
            pl.BlockSpec((D_C, D_C), lambda i: (0, 0)),
        ],
        out_specs=(
            pl.BlockSpec((T_C, NC, RPT, 32), lambda i: (0, 0, i, 0)),
            pl.BlockSpec((RPT, 8), lambda i: (i, 0)),
        ),
        out_shape=(
            jax.ShapeDtypeStruct((T_C, NC, NP, 32), jnp.float32),
            jax.ShapeDtypeStruct((NP, 8), jnp.float32),
        ),
    )(x, deg2, temporal, w0)


def _mid_body(v1, dinv, b0, w1, q2):
    dv = dinv[...][:, :1]
    for t in range(T_C):
        v = jnp.concatenate([v1[t, 0], v1[t, 1]], axis=1)
        h = jnp.maximum(dv * v + b0[...], 0.0)
        q = dv * jnp.dot(h, w1[...], preferred_element_type=jnp.float32,
                         precision=lax.Precision.HIGHEST)
        q2[t, 0] = q[:, :32]
        q2[t, 1] = q[:, 32:]


def _tc_mid(v1, dinv, b0, w1):
    nb = NP // RPT
    return pl.pallas_call(
        _mid_body,
        grid=(nb,),
        in_specs=[
            pl.BlockSpec((T_C, NC, RPT, 32), lambda i: (0, 0, i, 0)),
            pl.BlockSpec((RPT, 8), lambda i: (i, 0)),
            pl.BlockSpec((1, D_C), lambda i: (0, 0)),
            pl.BlockSpec((D_C, D_C), lambda i: (0, 0)),
        ],
        out_specs=pl.BlockSpec((T_C, NC, RPT, 32), lambda i: (0, 0, i, 0)),
        out_shape=jax.ShapeDtypeStruct((T_C, NC, NP, 32), jnp.float32),
    )(v1, dinv, b0, w1)


def _final_body(gu, gi, du, di, ue, ie, b1,
                wa1, ba1, wa2, ba2, wp1, bp1, wp2, bp2, out):
    dub = du[...][:, :1]
    dib = di[...][:, :1]
    b1v = b1[...]
    embs = []
    for t in range(T_C):
        guv = jnp.concatenate([gu[t, 0], gu[t, 1]], axis=1)
        giv = jnp.concatenate([gi[t, 0], gi[t, 1]], axis=1)
        ug = jnp.maximum(dub * guv + b1v, 0.0)
        ig = jnp.maximum(dib * giv + b1v, 0.0)
        embs.append(jnp.concatenate([ug, ig], axis=1))
    co = jnp.concatenate([ue[...], ie[...]], axis=1)
    ah = jnp.maximum(
        jnp.dot(co, wa1[...], preferred_element_type=jnp.float32,
                precision=lax.Precision.HIGHEST) + ba1[...],
        0.0)
    lg = jnp.dot(ah, wa2[...], preferred_element_type=jnp.float32,
                 precision=lax.Precision.HIGHEST) + ba2[...]
    m = jnp.max(lg, axis=1, keepdims=True)
    ex = jnp.exp(lg - m)
    att = ex / jnp.sum(ex, axis=1, keepdims=True)
    summed = att[:, 0:1] * embs[0]
    for t in range(1, T_C):
        summed = summed + att[:, t:t + 1] * embs[t]
    ph = jnp.maximum(
        jnp.dot(summed, wp1[...], preferred_element_type=jnp.float32,
                precision=lax.Precision.HIGHEST)
        + bp1[...], 0.0)
    out[...] = (jnp.dot(ph, wp2[...], preferred_element_type=jnp.float32,
                             precision=lax.Precision.HIGHEST)
                + bp2[...])


def _tc_final(gu, gi, du, di, ue, ie, b1, wa1, ba1, wa2, ba2,
              wp1, bp1, wp2, bp2):
    rb = 512
    nb = B_C // rb
    full = lambda i: (0, 0)
    return pl.pallas_call(
        _final_body,
        grid=(nb,),
        in_specs=[
            pl.BlockSpec((T_C, NC, rb, 32), lambda i: (0, 0, i, 0)),
            pl.BlockSpec((T_C, NC, rb, 32), lambda i: (0, 0, i, 0)),
            pl.BlockSpec((rb, 8), lambda i: (i, 0)),
            pl.BlockSpec((rb, 8), lambda i: (i, 0)),
            pl.BlockSpec((rb, D_C), lambda i: (i, 0)),
            pl.BlockSpec((rb, D_C), lambda i: (i, 0)),
            pl.BlockSpec((1, D_C), full),
            pl.BlockSpec((2 * D_C, D_C), full),
            pl.BlockSpec((1, D_C), full),
            pl.BlockSpec((D_C, T_C), full),
            pl.BlockSpec((1, T_C), full),
            pl.BlockSpec((2 * D_C, D_C), full),
            pl.BlockSpec((1, D_C), full),
            pl.BlockSpec((D_C, 1), full),
            pl.BlockSpec((1, 1), full),
        ],
        out_specs=pl.BlockSpec((rb, 1), lambda i: (i, 0)),
        out_shape=jax.ShapeDtypeStruct((B_C, 1), jnp.float32),
    )(gu, gi, du, di, ue, ie, b1, wa1, ba1, wa2, ba2, wp1, bp1, wp2, bp2)


def kernel(user_indices, item_indices, adj_row, adj_col, user_table,
           item_table, temporal, W0, b0, W1, b1, Wa1, ba1, Wa2, ba2,
           Wp1, bp1, Wp2, bp2):
    e = adj_row.shape[0]
    gtot = -(-e // (G * NS * K)) * NS * K
    epad = gtot * G
    row32 = adj_row.astype(jnp.int32)
    col32 = adj_col.astype(jnp.int32)
    rowg = jnp.concatenate(
        [row32, jnp.full((epad - e,), N_C, jnp.int32)]).reshape(gtot, G)
    colg = jnp.concatenate(
        [col32, jnp.zeros((epad - e,), jnp.int32)]).reshape(gtot, G)
    ui = user_indices.astype(jnp.int32)
    ii = item_indices.astype(jnp.int32)

    xpad = jnp.concatenate(
        [user_table, item_table,
         jnp.zeros((NP - N_C, D_C), jnp.float32)], axis=0)

    degp = _sc_deg(rowg).reshape(NC, NP)
    deg2 = (degp[0] + degp[1]).reshape(NP, 1)

    q1, dinv = _tc_prep(xpad, deg2, temporal, W0)
    v1 = _sc_spmm(colg, rowg, q1)
    q2 = _tc_mid(v1, dinv, b0.reshape(1, D_C), W1)
    v2 = _sc_spmm(colg, rowg, q2)
    gu, gi, du, di, ue, ie = _sc_gather(ui, ii, v2, dinv, user_table,
                                        item_table)
    out = _tc_final(gu, gi, du, di, ue, ie, b1.reshape(1, D_C),
                    Wa1, ba1.reshape(1, D_C), Wa2, ba2.reshape(1, T_C),
                    Wp1, bp1.reshape(1, D_C), Wp2, bp2.reshape(1, 1))
    return out.reshape(B_C)


# speedup vs baseline: 20.9862x; 1.3246x over previous
import functools

import jax
import jax.numpy as jnp
from jax import lax
from jax.experimental import pallas as pl
from jax.experimental.pallas import tpu as pltpu
from jax.experimental.pallas import tpu_sc as plsc

NU_C = 25000
NI_C = 25000
N_C = NU_C + NI_C
D_C = 64
T_C = 3
B_C = 4096

NC = 2
NS = 16
G = 128
K = 4

NP = 50048
RPT = NP // NS


def _mesh():
    return plsc.VectorSubcoreMesh(
        core_axis_name="c", subcore_axis_name="s", num_cores=NC, num_subcores=NS
    )


def _deg_body(gpt, n_chunk, rowg, degp, rowbuf, ones, zbuf, acc, sem):
    c = lax.axis_index("c")
    s = lax.axis_index("s")
    z16 = jnp.zeros((16,), jnp.float32)
    o16 = jnp.ones((16,), jnp.float32)

    def fill(i, _):
        zbuf[pl.ds(i * 16, 16)] = z16
        return 0

    lax.fori_loop(0, 64, fill, 0)
    for i in range(G // 16):
        ones[pl.ds(i * 16, 16)] = o16
    base = s * RPT
    for off in range(0, RPT, 1024):
        w = min(1024, RPT - off)
        pltpu.sync_copy(zbuf.at[pl.ds(0, w)], acc.at[pl.ds(base + off, w)])
    plsc.subcore_barrier()

    g0 = (c * NS + s) * gpt

    def chunk(i, _):
        pltpu.sync_copy(rowg.at[pl.ds(g0 + i * K, K)], rowbuf)
        descs = [
            pltpu.async_copy(ones.at[pl.ds(0, G)], acc.at[rowbuf.at[j]], sem,
                             add=True)
            for j in range(K)
        ]
        for dsc in descs:
            dsc.wait()
        return 0

    lax.fori_loop(0, n_chunk, chunk, 0)
    plsc.subcore_barrier()
    wo = 3072
    pltpu.sync_copy(acc.at[pl.ds(s * wo, wo)],
                    degp.at[pl.ds(c * NP + s * wo, wo)])
    tail = NP - NS * wo
    if tail:
        @pl.when(s == NS - 1)
        def _():
            pltpu.sync_copy(acc.at[pl.ds(NS * wo, tail)],
                            degp.at[pl.ds(c * NP + NS * wo, tail)])


def _sc_deg(rowg):
    gtot = rowg.shape[0]
    gpt = gtot // (NC * NS)
    body = functools.partial(_deg_body, gpt, gpt // K)
    return pl.kernel(
        body,
        out_type=jax.ShapeDtypeStruct((NC * NP,), jnp.float32),
        mesh=_mesh(),
        compiler_params=pltpu.CompilerParams(use_tc_tiling_on_sc=False),
        scratch_types=[
            pltpu.VMEM((K, G), jnp.int32),
            pltpu.VMEM((G,), jnp.float32),
            pltpu.VMEM((1024,), jnp.float32),
            pltpu.VMEM_SHARED((NP,), jnp.float32),
            pltpu.SemaphoreType.DMA,
        ],
    )(rowg)


def _spmm_body(tc, gpt, n_chunk, colg, rowg, q, out,
               colbuf, rowbuf, gbuf, acc, semg, sems):
    c = lax.axis_index("c")
    s = lax.axis_index("s")
    z16 = jnp.zeros((16,), jnp.float32)

    def fill(i, _):
        gbuf[i, pl.ds(0, 16)] = z16
        gbuf[i, pl.ds(16, 16)] = z16
        return 0

    lax.fori_loop(0, K * G, fill, 0)
    base = s * RPT
    g0 = s * gpt

    for t in range(tc):
        for off in range(0, RPT, K * G):
            w = min(K * G, RPT - off)
            pltpu.sync_copy(gbuf.at[pl.ds(0, w)], acc.at[pl.ds(base + off, w)])
        plsc.subcore_barrier()

        def chunk(i, _):
            pltpu.sync_copy(colg.at[pl.ds(g0 + i * K, K)], colbuf)
            pltpu.sync_copy(rowg.at[pl.ds(g0 + i * K, K)], rowbuf)
            gd = [
                pltpu.async_copy(q.at[t, c].at[colbuf.at[j]],
                                 gbuf.at[pl.ds(j * G, G)], semg)
                for j in range(K)
            ]
            for d in gd:
                d.wait()
            sd = [
                pltpu.async_copy(gbuf.at[pl.ds(j * G, G)],
                                 acc.at[rowbuf.at[j]], sems, add=True)
                for j in range(K)
            ]
            for d in sd:
                d.wait()
            return 0

        lax.fori_loop(0, n_chunk, chunk, 0)
        plsc.subcore_barrier()
        pltpu.sync_copy(acc.at[pl.ds(base, RPT)],
                        out.at[t, c, pl.ds(base, RPT)])
        if t + 1 < tc:
            lax.fori_loop(0, K * G, fill, 0)
            plsc.subcore_barrier()


def _sc_spmm(colg, rowg, q):
    tc = q.shape[0]
    gtot = colg.shape[0]
    gpt = gtot // NS
    body = functools.partial(_spmm_body, tc, gpt, gpt // K)
    return pl.kernel(
        body,
        out_type=jax.ShapeDtypeStruct((tc, NC, NP, 32), jnp.float32),
        mesh=_mesh(),
        compiler_params=pltpu.CompilerParams(use_tc_tiling_on_sc=False),
        scratch_types=[
            pltpu.VMEM((K, G), jnp.int32),
            pltpu.VMEM((K, G), jnp.int32),
            pltpu.VMEM((K * G, 32), jnp.float32),
            pltpu.VMEM_SHARED((NP, 32), jnp.float32),
            pltpu.SemaphoreType.DMA,
            pltpu.SemaphoreType.DMA,
        ],
    )(colg, rowg, q)


def _spmm16_body(gpt, n_chunk, colg, rowg, dinv16, out,
                 colbuf, rowbuf, gbuf, acc, semg, sems):
    c = lax.axis_index("c")
    s = lax.axis_index("s")
    z16 = jnp.zeros((16,), jnp.float32)

    def fill(i, _):
        gbuf[i, pl.ds(0, 16)] = z16
        return 0

    lax.fori_loop(0, K * G, fill, 0)
    base = s * RPT
    for off in range(0, RPT, K * G):
        w = min(K * G, RPT - off)
        pltpu.sync_copy(gbuf.at[pl.ds(0, w)], acc.at[pl.ds(base + off, w)])
    plsc.subcore_barrier()

    g0 = (c * NS + s) * gpt

    def chunk(i, _):
        pltpu.sync_copy(colg.at[pl.ds(g0 + i * K, K)], colbuf)
        pltpu.sync_copy(rowg.at[pl.ds(g0 + i * K, K)], rowbuf)
        gd = [
            pltpu.async_copy(dinv16.at[colbuf.at[j]],
                             gbuf.at[pl.ds(j * G, G)], semg)
            for j in range(K)
        ]
        for d in gd:
            d.wait()
        sd = [
            pltpu.async_copy(gbuf.at[pl.ds(j * G, G)],
                             acc.at[rowbuf.at[j]], sems, add=True)
            for j in range(K)
        ]
        for d in sd:
            d.wait()
        return 0

    lax.fori_loop(0, n_chunk, chunk, 0)
    plsc.subcore_barrier()
    pltpu.sync_copy(acc.at[pl.ds(base, RPT)],
                    out.at[c, pl.ds(base, RPT)])


def _sc_spmm16(colg, rowg, dinv16):
    gtot = colg.shape[0]
    gpt = gtot // (NC * NS)
    body = functools.partial(_spmm16_body, gpt, gpt // K)
    return pl.kernel(
        body,
        out_type=jax.ShapeDtypeStruct((NC, NP, 16), jnp.float32),
        mesh=_mesh(),
        compiler_params=pltpu.CompilerParams(use_tc_tiling_on_sc=False),
        scratch_types=[
            pltpu.VMEM((K, G), jnp.int32),
            pltpu.VMEM((K, G), jnp.int32),
            pltpu.VMEM((K * G, 16), jnp.float32),
            pltpu.VMEM_SHARED((NP, 16), jnp.float32),
            pltpu.SemaphoreType.DMA,
            pltpu.SemaphoreType.DMA,
        ],
    )(colg, rowg, dinv16)


def _gather_body(ui, ii, v2, dinv, ut, it,
                 gu, gi, du, di, ue, ie,
                 ubuf, ibuf, vbuf, ebuf, dbuf, sem):
    c = lax.axis_index("c")
    s = lax.axis_index("s")
    w = s * NC + c
    bpw = B_C // (NC * NS)
    b0 = w * bpw
    pltpu.sync_copy(ui.at[pl.ds(b0, bpw)], ubuf.at[0])
    pltpu.sync_copy(ii.at[pl.ds(b0, bpw)], ibuf.at[0])
    pltpu.async_copy(ut.at[ubuf.at[0]], ebuf, sem).wait()
    pltpu.sync_copy(ebuf, ue.at[pl.ds(b0, bpw)])
    pltpu.async_copy(it.at[ibuf.at[0]], ebuf, sem).wait()
    pltpu.sync_copy(ebuf, ie.at[pl.ds(b0, bpw)])
    pltpu.async_copy(dinv.at[ubuf.at[0]], dbuf, sem).wait()
    pltpu.sync_copy(dbuf, du.at[pl.ds(b0, bpw)])
    nu16 = jnp.full((16,), NU_C, jnp.int32)
    for j in range(bpw // 16):
        ibuf[0, pl.ds(j * 16, 16)] = ibuf[0, pl.ds(j * 16, 16)] + nu16
    pltpu.async_copy(dinv.at[ibuf.at[0]], dbuf, sem).wait()
    pltpu.sync_copy(dbuf, di.at[pl.ds(b0, bpw)])
    for t in range(T_C):
        for h in range(2):
            pltpu.async_copy(v2.at[t, h].at[ubuf.at[0]], vbuf, sem).wait()
            pltpu.sync_copy(vbuf, gu.at[t, h, pl.ds(b0, bpw)])
            pltpu.async_copy(v2.at[t, h].at[ibuf.at[0]], vbuf, sem).wait()
            pltpu.sync_copy(vbuf, gi.at[t, h, pl.ds(b0, bpw)])


def _sc_gather(ui, ii, v2, dinv, ut, it):
    bpw = B_C // (NC * NS)
    return pl.kernel(
        _gather_body,
        out_type=(
            jax.ShapeDtypeStruct((T_C, NC, B_C, 32), jnp.float32),
            jax.ShapeDtypeStruct((T_C, NC, B_C, 32), jnp.float32),
            jax.ShapeDtypeStruct((B_C, 16), jnp.float32),
            jax.ShapeDtypeStruct((B_C, 16), jnp.float32),
            jax.ShapeDtypeStruct((B_C, D_C), jnp.float32),
            jax.ShapeDtypeStruct((B_C, D_C), jnp.float32),
        ),
        mesh=_mesh(),
        compiler_params=pltpu.CompilerParams(use_tc_tiling_on_sc=False),
        scratch_types=[
            pltpu.VMEM((1, bpw), jnp.int32),
            pltpu.VMEM((1, bpw), jnp.int32),
            pltpu.VMEM((bpw, 32), jnp.float32),
            pltpu.VMEM((bpw, D_C), jnp.float32),
            pltpu.VMEM((bpw, 16), jnp.float32),
            pltpu.SemaphoreType.DMA,
        ],
    )(ui, ii, v2, dinv, ut, it)


def _prep_body(x, deg, w0, q1, dinv):
    dv1 = jnp.where(deg[...][:, :1] > 0.0, lax.rsqrt(deg[...][:, :1]), 0.0)
    dinv[...] = jnp.broadcast_to(dv1, dinv.shape)
    xw = jnp.dot(x[...], w0[...], preferred_element_type=jnp.float32,
                 precision=lax.Precision.HIGHEST)
    p = dv1 * xw
    q1[0, 0] = p[:, :32]
    q1[0, 1] = p[:, 32:]


def _tc_prep(x, deg2, w0):
    nb = NP // RPT
    return pl.pallas_call(
        _prep_body,
        grid=(nb,),
        in_specs=[
            pl.BlockSpec((RPT, D_C), lambda i: (i, 0)),
            pl.BlockSpec((RPT, 1), lambda i: (i, 0)),
            pl.BlockSpec((D_C, D_C), lambda i: (0, 0)),
        ],
        out_specs=(
            pl.BlockSpec((1, NC, RPT, 32), lambda i: (0, 0, i, 0)),
            pl.BlockSpec((RPT, 16), lambda i: (i, 0)),
        ),
        out_shape=(
            jax.ShapeDtypeStruct((1, NC, NP, 32), jnp.float32),
            jax.ShapeDtypeStruct((NP, 16), jnp.float32),
        ),
    )(x, deg2, w0)


def _mid_body(s1, sd2, dinv, temporal, w0, b0, w1, q2):
    dv = dinv[...][:, :1]
    sdv = (sd2[0] + sd2[1])[:, :1]
    s64 = jnp.concatenate([s1[0, 0], s1[0, 1]], axis=1)
    tw = jnp.dot(temporal[...], w0[...], preferred_element_type=jnp.float32,
                 precision=lax.Precision.HIGHEST)
    for t in range(T_C):
        v = s64 + sdv * tw[t][None, :]
        h = jnp.maximum(dv * v + b0[...], 0.0)
        q = dv * jnp.dot(h, w1[...], preferred_element_type=jnp.float32,
                         precision=lax.Precision.HIGHEST)
        q2[t, 0] = q[:, :32]
        q2[t, 1] = q[:, 32:]


def _tc_mid(s1, sd2, dinv, temporal, w0, b0, w1):
    nb = NP // RPT
    return pl.pallas_call(
        _mid_body,
        grid=(nb,),
        in_specs=[
            pl.BlockSpec((1, NC, RPT, 32), lambda i: (0, 0, i, 0)),
            pl.BlockSpec((NC, RPT, 16), lambda i: (0, i, 0)),
            pl.BlockSpec((RPT, 16), lambda i: (i, 0)),
            pl.BlockSpec((T_C, D_C), lambda i: (0, 0)),
            pl.BlockSpec((D_C, D_C), lambda i: (0, 0)),
            pl.BlockSpec((1, D_C), lambda i: (0, 0)),
            pl.BlockSpec((D_C, D_C), lambda i: (0, 0)),
        ],
        out_specs=pl.BlockSpec((T_C, NC, RPT, 32), lambda i: (0, 0, i, 0)),
        out_shape=jax.ShapeDtypeStruct((T_C, NC, NP, 32), jnp.float32),
    )(s1, sd2, dinv, temporal, w0, b0, w1)


def _final_body(gu, gi, du, di, ue, ie, b1,
                wa1, ba1, wa2, ba2, wp1, bp1, wp2, bp2, out):
    dub = du[...][:, :1]
    dib = di[...][:, :1]
    b1v = b1[...]
    embs = []
    for t in range(T_C):
        guv = jnp.concatenate([gu[t, 0], gu[t, 1]], axis=1)
        giv = jnp.concatenate([gi[t, 0], gi[t, 1]], axis=1)
        ug = jnp.maximum(dub * guv + b1v, 0.0)
        ig = jnp.maximum(dib * giv + b1v, 0.0)
        embs.append(jnp.concatenate([ug, ig], axis=1))
    co = jnp.concatenate([ue[...], ie[...]], axis=1)
    ah = jnp.maximum(
        jnp.dot(co, wa1[...], preferred_element_type=jnp.float32,
                precision=lax.Precision.HIGHEST) + ba1[...],
        0.0)
    lg = jnp.dot(ah, wa2[...], preferred_element_type=jnp.float32,
                 precision=lax.Precision.HIGHEST) + ba2[...]
    m = jnp.max(lg, axis=1, keepdims=True)
    ex = jnp.exp(lg - m)
    att = ex / jnp.sum(ex, axis=1, keepdims=True)
    summed = att[:, 0:1] * embs[0]
    for t in range(1, T_C):
        summed = summed + att[:, t:t + 1] * embs[t]
    ph = jnp.maximum(
        jnp.dot(summed, wp1[...], preferred_element_type=jnp.float32,
                precision=lax.Precision.HIGHEST)
        + bp1[...], 0.0)
    out[...] = (jnp.dot(ph, wp2[...], preferred_element_type=jnp.float32,
                             precision=lax.Precision.HIGHEST)
                + bp2[...])


def _tc_final(gu, gi, du, di, ue, ie, b1, wa1, ba1, wa2, ba2,
              wp1, bp1, wp2, bp2):
    rb = 512
    nb = B_C // rb
    full = lambda i: (0, 0)
    return pl.pallas_call(
        _final_body,
        grid=(nb,),
        in_specs=[
            pl.BlockSpec((T_C, NC, rb, 32), lambda i: (0, 0, i, 0)),
            pl.BlockSpec((T_C, NC, rb, 32), lambda i: (0, 0, i, 0)),
            pl.BlockSpec((rb, 16), lambda i: (i, 0)),
            pl.BlockSpec((rb, 16), lambda i: (i, 0)),
            pl.BlockSpec((rb, D_C), lambda i: (i, 0)),
            pl.BlockSpec((rb, D_C), lambda i: (i, 0)),
            pl.BlockSpec((1, D_C), full),
            pl.BlockSpec((2 * D_C, D_C), full),
            pl.BlockSpec((1, D_C), full),
            pl.BlockSpec((D_C, T_C), full),
            pl.BlockSpec((1, T_C), full),
            pl.BlockSpec((2 * D_C, D_C), full),
            pl.BlockSpec((1, D_C), full),
            pl.BlockSpec((D_C, 1), full),
            pl.BlockSpec((1, 1), full),
        ],
        out_specs=pl.BlockSpec((rb, 1), lambda i: (i, 0)),
        out_shape=jax.ShapeDtypeStruct((B_C, 1), jnp.float32),
    )(gu, gi, du, di, ue, ie, b1, wa1, ba1, wa2, ba2, wp1, bp1, wp2, bp2)


def kernel(user_indices, item_indices, adj_row, adj_col, user_table,
           item_table, temporal, W0, b0, W1, b1, Wa1, ba1, Wa2, ba2,
           Wp1, bp1, Wp2, bp2):
    e = adj_row.shape[0]
    gtot = -(-e // (G * NS * K)) * NS * K
    epad = gtot * G
    row32 = adj_row.astype(jnp.int32)
    col32 = adj_col.astype(jnp.int32)
    rowg = jnp.concatenate(
        [row32, jnp.full((epad - e,), N_C, jnp.int32)]).reshape(gtot, G)
    colg = jnp.concatenate(
        [col32, jnp.zeros((epad - e,), jnp.int32)]).reshape(gtot, G)
    ui = user_indices.astype(jnp.int32)
    ii = item_indices.astype(jnp.int32)

    xpad = jnp.concatenate(
        [user_table, item_table,
         jnp.zeros((NP - N_C, D_C), jnp.float32)], axis=0)

    degp = _sc_deg(rowg).reshape(NC, NP)
    deg2 = (degp[0] + degp[1]).reshape(NP, 1)

    q1, dinv = _tc_prep(xpad, deg2, W0)
    s1 = _sc_spmm(colg, rowg, q1)
    sd2 = _sc_spmm16(colg, rowg, dinv)
    q2 = _tc_mid(s1, sd2, dinv, temporal, W0, b0.reshape(1, D_C), W1)
    v2 = _sc_spmm(colg, rowg, q2)
    gu, gi, du, di, ue, ie = _sc_gather(ui, ii, v2, dinv, user_table,
                                        item_table)
    out = _tc_final(gu, gi, du, di, ue, ie, b1.reshape(1, D_C),
                    Wa1, ba1.reshape(1, D_C), Wa2, ba2.reshape(1, T_C),
                    Wp1, bp1.reshape(1, D_C), Wp2, bp2.reshape(1, 1))
    return out.reshape(B_C)
